# Initial kernel scaffold; baseline (speedup 1.0000x reference)
import jax
import jax.numpy as jnp
from jax.experimental import pallas as pl


def kernel(x, batch):
    raise NotImplementedError("write your pallas kernel here")


# speedup vs baseline: 4.8930x; 4.8930x over previous
import functools

import jax
import jax.numpy as jnp
from jax import lax
from jax.experimental import pallas as pl
from jax.experimental.pallas import tpu as pltpu
from jax.experimental.pallas import tpu_sc as plsc

N_ROWS = 320000
D_FEAT = 128
NUM_SEGMENTS = 10000
NV = D_FEAT // 16
NW = 32
SPT = 313
S_PAD = NW * SPT
CHUNK = 256
STAGE = SPT * D_FEAT

def _lane(vec, lane):
    idxv = (lax.iota(jnp.int32, 16) + lane) & 15
    return vec.at[idxv].get(mode="promise_in_bounds")[0]


def _searchsorted(b_hbm, probe, target):
    def step(_, lohi):
        lo, hi = lohi
        mid = jnp.maximum((lo + hi) // 2, 0)
        off = jnp.minimum((mid // 8) * 8, N_ROWS - 16)
        pltpu.sync_copy(b_hbm.at[pl.ds(off, 16)], probe)
        ge = _lane(probe[...], mid - off) >= target
        return (jnp.where(ge, lo, mid), jnp.where(ge, mid, hi))

    _, hi = lax.fori_loop(0, 19, step, (jnp.int32(-1), jnp.int32(N_ROWS)))
    return hi


def _body(xf_hbm, b_hbm, om_hbm, ox_hbm, xbuf, bbuf, probe, mst, xst):
    wid = lax.axis_index("c") * 16 + lax.axis_index("s")
    seg_lo = wid * SPT

    zero = jnp.zeros((16,), jnp.float32)

    def zstep(j, carry):
        mst[pl.ds(j * 16, 16)] = zero
        xst[pl.ds(j * 16, 16)] = zero
        return carry

    lax.fori_loop(0, SPT * NV, zstep, 0)

    r_lo = _searchsorted(b_hbm, probe, seg_lo)
    r_hi = _searchsorted(b_hbm, probe, seg_lo + SPT)

    def flush(seg_prev, cnt, sums, maxs):
        base = (seg_prev - seg_lo) * D_FEAT
        inv = 1.0 / jnp.full((16,), cnt, jnp.int32).astype(jnp.float32)
        for k in range(NV):
            mst[pl.ds(base + k * 16, 16)] = sums[k] * inv
            xst[pl.ds(base + k * 16, 16)] = maxs[k]

    def row_body(i, carry):
        seg_prev, cnt = carry[0], carry[1]
        sums, maxs = carry[2:2 + NV], carry[2 + NV:]
        seg = bbuf[pl.ds(i, 16)][0]
        is_new = seg != seg_prev

        @pl.when(jnp.logical_and(is_new, seg_prev >= 0))
        def _():
            flush(seg_prev, cnt, sums, maxs)

        d = jnp.where(is_new, jnp.int32(1), jnp.int32(0))
        keep = jnp.full((16,), 1 - d, jnp.int32).astype(jnp.float32)
        pen = jnp.full((16,), d, jnp.int32).astype(jnp.float32) * (-3.4e38)
        xk = [xbuf[pl.ds(i * D_FEAT + k * 16, 16)] for k in range(NV)]
        nsums = [xk[k] + keep * sums[k] for k in range(NV)]
        nmaxs = [jnp.maximum(xk[k], maxs[k] * keep + pen) for k in range(NV)]
        ncnt = jnp.where(is_new, jnp.int32(1), cnt + 1)
        return (seg, ncnt) + tuple(nsums) + tuple(nmaxs)

    def chunk_body(k, carry):
        pltpu.sync_copy(xf_hbm.at[pl.ds(k * (CHUNK * D_FEAT), CHUNK * D_FEAT)],
                        xbuf)
        pltpu.sync_copy(b_hbm.at[pl.ds(k * CHUNK, CHUNK)],
                        bbuf.at[pl.ds(0, CHUNK)])
        a = jnp.maximum(r_lo - k * CHUNK, 0)
        b = jnp.minimum(r_hi - k * CHUNK, CHUNK)
        return lax.fori_loop(a, b, row_body, carry)

    init = ((jnp.int32(-1), jnp.int32(0))
            + tuple(jnp.zeros((16,), jnp.float32) for _ in range(2 * NV)))
    k_lo = r_lo // CHUNK
    k_hi = (r_hi + CHUNK - 1) // CHUNK
    final = lax.fori_loop(k_lo, k_hi, chunk_body, init)
    seg_prev, cnt = final[0], final[1]

    @pl.when(seg_prev >= 0)
    def _():
        flush(seg_prev, cnt, final[2:2 + NV], final[2 + NV:])

    pltpu.sync_copy(mst, om_hbm.at[pl.ds(seg_lo * D_FEAT, STAGE)])
    pltpu.sync_copy(xst, ox_hbm.at[pl.ds(seg_lo * D_FEAT, STAGE)])


def _make_pool():
    return functools.partial(
        pl.kernel,
        out_type=[jax.ShapeDtypeStruct((S_PAD * D_FEAT,), jnp.float32),
                  jax.ShapeDtypeStruct((S_PAD * D_FEAT,), jnp.float32)],
        mesh=plsc.VectorSubcoreMesh(core_axis_name="c", subcore_axis_name="s"),
        scratch_types=[
            pltpu.VMEM((CHUNK * D_FEAT,), jnp.float32),
            pltpu.VMEM((CHUNK + 16,), jnp.int32),
            pltpu.VMEM((16,), jnp.int32),
            pltpu.VMEM((STAGE,), jnp.float32),
            pltpu.VMEM((STAGE,), jnp.float32),
        ],
    )(_body)


def kernel(x, batch):
    om, ox = _make_pool()(x.reshape(-1), batch)
    mean = om.reshape(S_PAD, D_FEAT)[:NUM_SEGMENTS]
    mx = ox.reshape(S_PAD, D_FEAT)[:NUM_SEGMENTS]
    return jnp.concatenate([mean, mx], axis=-1)


# speedup vs baseline: 6.3893x; 1.3058x over previous
import functools

import jax
import jax.numpy as jnp
from jax import lax
from jax.experimental import pallas as pl
from jax.experimental.pallas import tpu as pltpu
from jax.experimental.pallas import tpu_sc as plsc

N_ROWS = 320000
D_FEAT = 128
NUM_SEGMENTS = 10000
NV = D_FEAT // 16
NW = 32
SPT = 313
S_PAD = NW * SPT
CHUNK = 128
CD = CHUNK * D_FEAT
STAGE = SPT * D_FEAT


def _lane(vec, lane):
    idxv = (lax.iota(jnp.int32, 16) + lane) & 15
    return vec.at[idxv].get(mode="promise_in_bounds")[0]


def _search2(b_hbm, pa, pb, sema, semb, ta, tb):
    def probe_off(lo, hi):
        mid = jnp.maximum((lo + hi) // 2, 0)
        off = jnp.minimum((mid // 8) * 8, N_ROWS - 16)
        return mid, off

    def step(_, st):
        loa, hia, lob, hib = st
        mida, offa = probe_off(loa, hia)
        midb, offb = probe_off(lob, hib)
        cpa = pltpu.make_async_copy(b_hbm.at[pl.ds(offa, 16)], pa, sema)
        cpb = pltpu.make_async_copy(b_hbm.at[pl.ds(offb, 16)], pb, semb)
        cpa.start()
        cpb.start()
        cpa.wait()
        cpb.wait()
        gea = _lane(pa[...], mida - offa) >= ta
        geb = _lane(pb[...], midb - offb) >= tb
        return (jnp.where(gea, loa, mida), jnp.where(gea, mida, hia),
                jnp.where(geb, lob, midb), jnp.where(geb, midb, hib))

    init = (jnp.int32(-1), jnp.int32(N_ROWS), jnp.int32(-1), jnp.int32(N_ROWS))
    _, ra, _, rb = lax.fori_loop(0, 19, step, init)
    return ra, rb


def _body(xf_hbm, b_hbm, om_hbm, ox_hbm,
          xbuf, bbuf, pa, pb, mst, xst, sema, semb):
    wid = lax.axis_index("c") * 16 + lax.axis_index("s")
    seg_lo = wid * SPT

    zero = jnp.zeros((16,), jnp.float32)

    def zstep(j, carry):
        mst[pl.ds(j * 16, 16)] = zero
        xst[pl.ds(j * 16, 16)] = zero
        return carry

    lax.fori_loop(0, SPT * NV, zstep, 0)

    r_lo, r_hi = _search2(b_hbm, pa, pb, sema, semb, seg_lo, seg_lo + SPT)

    k_lo = r_lo // CHUNK
    k_hi = (r_hi + CHUNK - 1) // CHUNK

    def chunk_copies(k):
        p = k & 1
        cpx = pltpu.make_async_copy(
            xf_hbm.at[pl.ds(k * CD, CD)],
            xbuf.at[pl.ds(p * CD, CD)],
            sema)
        cpb = pltpu.make_async_copy(
            b_hbm.at[pl.ds(k * CHUNK, CHUNK)],
            bbuf.at[pl.ds(p * (CHUNK + 16), CHUNK)],
            semb)
        return cpx, cpb

    def issue(k):
        cpx, cpb = chunk_copies(k)
        cpx.start()
        cpb.start()

    def drain(k):
        cpx, cpb = chunk_copies(k)
        cpx.wait()
        cpb.wait()

    @pl.when(k_lo < k_hi)
    def _():
        issue(k_lo)

    def flush(seg_prev, cnt, sums, maxs):
        base = (seg_prev - seg_lo) * D_FEAT
        inv = 1.0 / jnp.full((16,), cnt, jnp.int32).astype(jnp.float32)
        for k in range(NV):
            mst[pl.ds(base + k * 16, 16)] = sums[k] * inv
            xst[pl.ds(base + k * 16, 16)] = maxs[k]

    def make_row_body(xoff, boff):
        def row_body(i, carry):
            seg_prev, cnt = carry[0], carry[1]
            sums, maxs = carry[2:2 + NV], carry[2 + NV:]
            seg = bbuf[pl.ds(boff + i, 16)][0]
            is_new = seg != seg_prev

            @pl.when(jnp.logical_and(is_new, seg_prev >= 0))
            def _():
                flush(seg_prev, cnt, sums, maxs)

            d = jnp.where(is_new, jnp.int32(1), jnp.int32(0))
            keep = jnp.full((16,), 1 - d, jnp.int32).astype(jnp.float32)
            pen = jnp.full((16,), d, jnp.int32).astype(jnp.float32) * (-3.4e38)
            xk = [xbuf[pl.ds(xoff + i * D_FEAT + k * 16, 16)]
                  for k in range(NV)]
            nsums = [xk[k] + keep * sums[k] for k in range(NV)]
            nmaxs = [jnp.maximum(xk[k], maxs[k] * keep + pen)
                     for k in range(NV)]
            ncnt = jnp.where(is_new, jnp.int32(1), cnt + 1)
            return (seg, ncnt) + tuple(nsums) + tuple(nmaxs)
        return row_body

    def chunk_body(k, carry):
        drain(k)

        @pl.when(k + 1 < k_hi)
        def _():
            issue(k + 1)

        p = k & 1
        a = jnp.maximum(r_lo - k * CHUNK, 0)
        b = jnp.minimum(r_hi - k * CHUNK, CHUNK)
        return lax.fori_loop(a, b,
                             make_row_body(p * CD, p * (CHUNK + 16)), carry)

    init = ((jnp.int32(-1), jnp.int32(0))
            + tuple(jnp.zeros((16,), jnp.float32) for _ in range(2 * NV)))
    final = lax.fori_loop(k_lo, k_hi, chunk_body, init)
    seg_prev, cnt = final[0], final[1]

    @pl.when(seg_prev >= 0)
    def _():
        flush(seg_prev, cnt, final[2:2 + NV], final[2 + NV:])

    pltpu.sync_copy(mst, om_hbm.at[pl.ds(seg_lo * D_FEAT, STAGE)])
    pltpu.sync_copy(xst, ox_hbm.at[pl.ds(seg_lo * D_FEAT, STAGE)])


def _make_pool():
    return functools.partial(
        pl.kernel,
        out_type=[jax.ShapeDtypeStruct((S_PAD * D_FEAT,), jnp.float32),
                  jax.ShapeDtypeStruct((S_PAD * D_FEAT,), jnp.float32)],
        mesh=plsc.VectorSubcoreMesh(core_axis_name="c", subcore_axis_name="s"),
        scratch_types=[
            pltpu.VMEM((2 * CD,), jnp.float32),
            pltpu.VMEM((2 * (CHUNK + 16),), jnp.int32),
            pltpu.VMEM((16,), jnp.int32),
            pltpu.VMEM((16,), jnp.int32),
            pltpu.VMEM((STAGE,), jnp.float32),
            pltpu.VMEM((STAGE,), jnp.float32),
            pltpu.SemaphoreType.DMA,
            pltpu.SemaphoreType.DMA,
        ],
    )(_body)


def kernel(x, batch):
    om, ox = _make_pool()(x.reshape(-1), batch)
    mean = om.reshape(S_PAD, D_FEAT)[:NUM_SEGMENTS]
    mx = ox.reshape(S_PAD, D_FEAT)[:NUM_SEGMENTS]
    return jnp.concatenate([mean, mx], axis=-1)


# speedup vs baseline: 10.7108x; 1.6764x over previous
import functools

import jax
import jax.numpy as jnp
from jax import lax
from jax.experimental import pallas as pl
from jax.experimental.pallas import tpu as pltpu
from jax.experimental.pallas import tpu_sc as plsc

N_ROWS = 320000
D_FEAT = 128
NUM_SEGMENTS = 10000
NV = D_FEAT // 16
NW = 32
SPT = 313
S_PAD = NW * SPT
CHUNK = 128
CD = CHUNK * D_FEAT
STAGE = SPT * D_FEAT


def _lane(vec, lane):
    idxv = (lax.iota(jnp.int32, 16) + lane) & 15
    return vec.at[idxv].get(mode="promise_in_bounds")[0]


def _search2(b_hbm, pa, pb, sema, semb, ta, tb):
    def probe_off(lo, hi):
        mid = jnp.maximum((lo + hi) // 2, 0)
        off = jnp.minimum((mid // 8) * 8, N_ROWS - 16)
        return mid, off

    def step(_, st):
        loa, hia, lob, hib = st
        mida, offa = probe_off(loa, hia)
        midb, offb = probe_off(lob, hib)
        cpa = pltpu.make_async_copy(b_hbm.at[pl.ds(offa, 16)], pa, sema)
        cpb = pltpu.make_async_copy(b_hbm.at[pl.ds(offb, 16)], pb, semb)
        cpa.start()
        cpb.start()
        cpa.wait()
        cpb.wait()
        gea = _lane(pa[...], mida - offa) >= ta
        geb = _lane(pb[...], midb - offb) >= tb
        return (jnp.where(gea, loa, mida), jnp.where(gea, mida, hia),
                jnp.where(geb, lob, midb), jnp.where(geb, midb, hib))

    init = (jnp.int32(-1), jnp.int32(N_ROWS), jnp.int32(-1), jnp.int32(N_ROWS))
    _, ra, _, rb = lax.fori_loop(0, 19, step, init)
    return ra, rb


def _body(xf_hbm, b_hbm, om_hbm, ox_hbm,
          xbuf, bbuf, pa, pb, mst, xst, sema, semb):
    wid = lax.axis_index("c") * 16 + lax.axis_index("s")
    seg_lo = wid * SPT

    zero = jnp.zeros((16,), jnp.float32)

    def zstep(j, carry):
        mst[pl.ds(j * 16, 16)] = zero
        xst[pl.ds(j * 16, 16)] = zero
        return carry

    lax.fori_loop(0, SPT * NV, zstep, 0)

    r_lo, r_hi = _search2(b_hbm, pa, pb, sema, semb, seg_lo, seg_lo + SPT)

    k_lo = r_lo // CHUNK
    k_hi = (r_hi + CHUNK - 1) // CHUNK
    full_start = (r_lo + CHUNK - 1) // CHUNK
    full_end = r_hi // CHUNK
    head_hi = jnp.minimum(full_start * CHUNK, r_hi)
    tail_lo = jnp.maximum(full_end * CHUNK, head_hi)

    def chunk_copies(k):
        p = k & 1
        cpx = pltpu.make_async_copy(
            xf_hbm.at[pl.ds(k * CD, CD)],
            xbuf.at[pl.ds(p * CD, CD)],
            sema)
        cpb = pltpu.make_async_copy(
            b_hbm.at[pl.ds(k * CHUNK, CHUNK)],
            bbuf.at[pl.ds(p * (CHUNK + 16), CHUNK)],
            semb)
        return cpx, cpb

    def issue(k):
        cpx, cpb = chunk_copies(k)
        cpx.start()
        cpb.start()

    def drain(k):
        cpx, cpb = chunk_copies(k)
        cpx.wait()
        cpb.wait()

    @pl.when(r_lo < r_hi)
    def _():
        issue(k_lo)

    def flush(seg_prev, cnt, sums, maxs):
        base = (seg_prev - seg_lo) * D_FEAT
        inv = 1.0 / jnp.full((16,), cnt, jnp.int32).astype(jnp.float32)
        for k in range(NV):
            mst[pl.ds(base + k * 16, 16)] = sums[k] * inv
            xst[pl.ds(base + k * 16, 16)] = maxs[k]

    def merge_row(seg, xk, carry):
        seg_prev, cnt = carry[0], carry[1]
        sums, maxs = carry[2:2 + NV], carry[2 + NV:]
        is_new = seg != seg_prev

        @pl.when(jnp.logical_and(is_new, seg_prev >= 0))
        def _():
            flush(seg_prev, cnt, sums, maxs)

        d = jnp.where(is_new, jnp.int32(1), jnp.int32(0))
        keep = jnp.full((16,), 1 - d, jnp.int32).astype(jnp.float32)
        pen = jnp.full((16,), d, jnp.int32).astype(jnp.float32) * (-3.4e38)
        nsums = [xk[k] + keep * sums[k] for k in range(NV)]
        nmaxs = [jnp.maximum(xk[k], maxs[k] * keep + pen) for k in range(NV)]
        ncnt = jnp.where(is_new, jnp.int32(1), cnt + 1)
        return (seg, ncnt) + tuple(nsums) + tuple(nmaxs)

    def row_dyn(k):
        p = k & 1
        xoff = p * CD
        boff = p * (CHUNK + 16)

        def body(i, carry):
            seg = bbuf[pl.ds(boff + i, 16)][0]
            xk = [xbuf[pl.ds(xoff + i * D_FEAT + k2 * 16, 16)]
                  for k2 in range(NV)]
            return merge_row(seg, xk, carry)

        return body

    def chunk_body(k, carry):
        drain(k)

        @pl.when(k + 1 < k_hi)
        def _():
            issue(k + 1)

        p = k & 1
        xoff = p * CD
        boff = p * (CHUNK + 16)

        def group_body(g, c):
            i0 = g * 16
            bv = bbuf[pl.ds(boff + i0, 16)]
            for j in range(16):
                xk = [xbuf[pl.ds(xoff + (i0 + j) * D_FEAT + k2 * 16, 16)]
                      for k2 in range(NV)]
                c = merge_row(bv[j], xk, c)
            return c

        return lax.fori_loop(0, CHUNK // 16, group_body, carry)

    carry = ((jnp.int32(-1), jnp.int32(0))
             + tuple(jnp.zeros((16,), jnp.float32) for _ in range(2 * NV)))

    @pl.when(head_hi > r_lo)
    def _():
        drain(k_lo)

        @pl.when(k_lo + 1 < k_hi)
        def _():
            issue(k_lo + 1)

    carry = lax.fori_loop(r_lo - k_lo * CHUNK, head_hi - k_lo * CHUNK,
                          row_dyn(k_lo), carry)

    carry = lax.fori_loop(full_start, full_end, chunk_body, carry)

    @pl.when(r_hi > tail_lo)
    def _():
        drain(k_hi - 1)

    carry = lax.fori_loop(tail_lo - (k_hi - 1) * CHUNK,
                          r_hi - (k_hi - 1) * CHUNK,
                          row_dyn(k_hi - 1), carry)

    seg_prev, cnt = carry[0], carry[1]

    @pl.when(seg_prev >= 0)
    def _():
        flush(seg_prev, cnt, carry[2:2 + NV], carry[2 + NV:])

    pltpu.sync_copy(mst, om_hbm.at[pl.ds(seg_lo * D_FEAT, STAGE)])
    pltpu.sync_copy(xst, ox_hbm.at[pl.ds(seg_lo * D_FEAT, STAGE)])


def _make_pool():
    return functools.partial(
        pl.kernel,
        out_type=[jax.ShapeDtypeStruct((S_PAD * D_FEAT,), jnp.float32),
                  jax.ShapeDtypeStruct((S_PAD * D_FEAT,), jnp.float32)],
        mesh=plsc.VectorSubcoreMesh(core_axis_name="c", subcore_axis_name="s"),
        scratch_types=[
            pltpu.VMEM((2 * CD,), jnp.float32),
            pltpu.VMEM((2 * (CHUNK + 16),), jnp.int32),
            pltpu.VMEM((16,), jnp.int32),
            pltpu.VMEM((16,), jnp.int32),
            pltpu.VMEM((STAGE,), jnp.float32),
            pltpu.VMEM((STAGE,), jnp.float32),
            pltpu.SemaphoreType.DMA,
            pltpu.SemaphoreType.DMA,
        ],
    )(_body)


def kernel(x, batch):
    om, ox = _make_pool()(x.reshape(-1), batch)
    mean = om.reshape(S_PAD, D_FEAT)[:NUM_SEGMENTS]
    mx = ox.reshape(S_PAD, D_FEAT)[:NUM_SEGMENTS]
    return jnp.concatenate([mean, mx], axis=-1)


# speedup vs baseline: 10.7415x; 1.0029x over previous
import functools

import jax
import jax.numpy as jnp
from jax import lax
from jax.experimental import pallas as pl
from jax.experimental.pallas import tpu as pltpu
from jax.experimental.pallas import tpu_sc as plsc

N_ROWS = 320000
D_FEAT = 128
NUM_SEGMENTS = 10000
NV = D_FEAT // 16
NW = 32
SPT = 313
S_PAD = NW * SPT
CHUNK = 128
CD = CHUNK * D_FEAT
STAGE = SPT * D_FEAT


def _lane(vec, lane):
    idxv = (lax.iota(jnp.int32, 16) + lane) & 15
    return vec.at[idxv].get(mode="promise_in_bounds")[0]


def _search2(b_hbm, pa, pb, sema, semb, ta, tb):
    def probe_off(lo, hi):
        mid = jnp.maximum((lo + hi) // 2, 0)
        off = jnp.minimum((mid // 8) * 8, N_ROWS - 16)
        return mid, off

    def step(_, st):
        loa, hia, lob, hib = st
        mida, offa = probe_off(loa, hia)
        midb, offb = probe_off(lob, hib)
        cpa = pltpu.make_async_copy(b_hbm.at[pl.ds(offa, 16)], pa, sema)
        cpb = pltpu.make_async_copy(b_hbm.at[pl.ds(offb, 16)], pb, semb)
        cpa.start()
        cpb.start()
        cpa.wait()
        cpb.wait()
        gea = _lane(pa[...], mida - offa) >= ta
        geb = _lane(pb[...], midb - offb) >= tb
        return (jnp.where(gea, loa, mida), jnp.where(gea, mida, hia),
                jnp.where(geb, lob, midb), jnp.where(geb, midb, hib))

    init = (jnp.int32(-1), jnp.int32(N_ROWS), jnp.int32(-1), jnp.int32(N_ROWS))
    _, ra, _, rb = lax.fori_loop(0, 19, step, init)
    return ra, rb


def _body(xf_hbm, b_hbm, om_hbm, ox_hbm,
          xbuf, bbuf, pa, pb, mst, xst, sema, semb):
    wid = lax.axis_index("c") * 16 + lax.axis_index("s")
    seg_lo = wid * SPT

    zero = jnp.zeros((16,), jnp.float32)

    def zstep(j, carry):
        mst[pl.ds(j * 16, 16)] = zero
        xst[pl.ds(j * 16, 16)] = zero
        return carry

    lax.fori_loop(0, SPT * NV, zstep, 0)

    r_lo, r_hi = _search2(b_hbm, pa, pb, sema, semb, seg_lo, seg_lo + SPT)

    off0 = jnp.minimum((r_lo // 8) * 8, N_ROWS - 16)
    cp0 = pltpu.make_async_copy(b_hbm.at[pl.ds(off0, 16)], pa, sema)
    cp0.start()
    cp0.wait()
    seg0 = _lane(pa[...], r_lo - off0)

    k_lo = r_lo // CHUNK
    k_hi = (r_hi + CHUNK - 1) // CHUNK
    full_start = (r_lo + CHUNK - 1) // CHUNK
    full_end = r_hi // CHUNK
    head_hi = jnp.minimum(full_start * CHUNK, r_hi)
    tail_lo = jnp.maximum(full_end * CHUNK, head_hi)

    def chunk_copies(k):
        p = k & 1
        cpx = pltpu.make_async_copy(
            xf_hbm.at[pl.ds(k * CD, CD)],
            xbuf.at[pl.ds(p * CD, CD)],
            sema)
        cpb = pltpu.make_async_copy(
            b_hbm.at[pl.ds(k * CHUNK, CHUNK)],
            bbuf.at[pl.ds(p * (CHUNK + 16), CHUNK)],
            semb)
        return cpx, cpb

    def issue(k):
        cpx, cpb = chunk_copies(k)
        cpx.start()
        cpb.start()

    def drain(k):
        cpx, cpb = chunk_copies(k)
        cpx.wait()
        cpb.wait()

    @pl.when(r_lo < r_hi)
    def _():
        issue(k_lo)

    def flush(seg_prev, cnt, sums, maxs):
        base = (seg_prev - seg_lo) * D_FEAT
        inv = 1.0 / jnp.full((16,), cnt, jnp.int32).astype(jnp.float32)
        for k in range(NV):
            mst[pl.ds(base + k * 16, 16)] = sums[k] * inv
            xst[pl.ds(base + k * 16, 16)] = maxs[k]

    def merge_row(seg, xk, carry):
        seg_prev, cnt = carry[0], carry[1]
        sums, maxs = carry[2:2 + NV], carry[2 + NV:]
        is_new = seg != seg_prev

        @pl.when(is_new)
        def _():
            flush(seg_prev, cnt, sums, maxs)

        keep = jnp.full((16,), jnp.where(is_new, 0.0, 1.0), jnp.float32)
        pen = keep * jnp.float32(3.4e38) - jnp.float32(3.4e38)
        nsums = [xk[k] + keep * sums[k] for k in range(NV)]
        nmaxs = [jnp.maximum(xk[k], maxs[k] * keep + pen) for k in range(NV)]
        ncnt = jnp.where(is_new, jnp.int32(1), cnt + 1)
        return (seg, ncnt) + tuple(nsums) + tuple(nmaxs)

    def row_dyn(k):
        p = k & 1
        xoff = p * CD
        boff = p * (CHUNK + 16)

        def body(i, carry):
            seg = bbuf[pl.ds(boff + i, 16)][0]
            xk = [xbuf[pl.ds(xoff + i * D_FEAT + k2 * 16, 16)]
                  for k2 in range(NV)]
            return merge_row(seg, xk, carry)

        return body

    def chunk_body(k, carry):
        drain(k)

        @pl.when(k + 1 < k_hi)
        def _():
            issue(k + 1)

        p = k & 1
        xoff = p * CD
        boff = p * (CHUNK + 16)

        def group_body(g, c):
            i0 = g * 16
            bv = bbuf[pl.ds(boff + i0, 16)]
            for j in range(16):
                xk = [xbuf[pl.ds(xoff + (i0 + j) * D_FEAT + k2 * 16, 16)]
                      for k2 in range(NV)]
                c = merge_row(bv[j], xk, c)
            return c

        return lax.fori_loop(0, CHUNK // 16, group_body, carry)

    neg = jnp.full((16,), -3.4e38, jnp.float32)
    carry = ((seg0, jnp.int32(0))
             + tuple(jnp.zeros((16,), jnp.float32) for _ in range(NV))
             + tuple(neg for _ in range(NV)))

    @pl.when(head_hi > r_lo)
    def _():
        drain(k_lo)

        @pl.when(k_lo + 1 < k_hi)
        def _():
            issue(k_lo + 1)

    carry = lax.fori_loop(r_lo - k_lo * CHUNK, head_hi - k_lo * CHUNK,
                          row_dyn(k_lo), carry)

    carry = lax.fori_loop(full_start, full_end, chunk_body, carry)

    @pl.when(r_hi > tail_lo)
    def _():
        drain(k_hi - 1)

    carry = lax.fori_loop(tail_lo - (k_hi - 1) * CHUNK,
                          r_hi - (k_hi - 1) * CHUNK,
                          row_dyn(k_hi - 1), carry)

    seg_prev, cnt = carry[0], carry[1]

    @pl.when(cnt > 0)
    def _():
        flush(seg_prev, cnt, carry[2:2 + NV], carry[2 + NV:])

    pltpu.sync_copy(mst, om_hbm.at[pl.ds(seg_lo * D_FEAT, STAGE)])
    pltpu.sync_copy(xst, ox_hbm.at[pl.ds(seg_lo * D_FEAT, STAGE)])


def _make_pool():
    return functools.partial(
        pl.kernel,
        out_type=[jax.ShapeDtypeStruct((S_PAD * D_FEAT,), jnp.float32),
                  jax.ShapeDtypeStruct((S_PAD * D_FEAT,), jnp.float32)],
        mesh=plsc.VectorSubcoreMesh(core_axis_name="c", subcore_axis_name="s"),
        scratch_types=[
            pltpu.VMEM((2 * CD,), jnp.float32),
            pltpu.VMEM((2 * (CHUNK + 16),), jnp.int32),
            pltpu.VMEM((16,), jnp.int32),
            pltpu.VMEM((16,), jnp.int32),
            pltpu.VMEM((STAGE,), jnp.float32),
            pltpu.VMEM((STAGE,), jnp.float32),
            pltpu.SemaphoreType.DMA,
            pltpu.SemaphoreType.DMA,
        ],
    )(_body)


def kernel(x, batch):
    om, ox = _make_pool()(x.reshape(-1), batch)
    mean = om.reshape(S_PAD, D_FEAT)[:NUM_SEGMENTS]
    mx = ox.reshape(S_PAD, D_FEAT)[:NUM_SEGMENTS]
    return jnp.concatenate([mean, mx], axis=-1)
